# speedup vs baseline: 4.4456x; 4.4456x over previous
import functools

import jax
import jax.numpy as jnp
from jax import lax
from jax.experimental import pallas as pl
from jax.experimental.pallas import tpu as pltpu
from jax.experimental.pallas import tpu_sc as plsc

D = 128
CHUNK = 128
NBUF = 4
NC = 2
NS = 16


@functools.cache
def _make_emb(n_rows):
    n_chunks = n_rows // CHUNK
    nw = NC * NS
    ch_per_w = n_chunks // nw
    n_outer = ch_per_w // NBUF
    assert n_chunks % nw == 0 and ch_per_w % NBUF == 0

    mesh = plsc.VectorSubcoreMesh(core_axis_name="c", subcore_axis_name="s")

    @functools.partial(
        pl.kernel,
        out_type=jax.ShapeDtypeStruct((n_rows, D), jnp.float32),
        mesh=mesh,
        scratch_types=[
            pltpu.VMEM((ch_per_w, CHUNK), jnp.int32),
            *[pltpu.VMEM((CHUNK, D), jnp.float32) for _ in range(NBUF)],
            *[pltpu.SemaphoreType.DMA for _ in range(2 * NBUF)],
        ],
    )
    def emb(table_hbm, idx_hbm, out_hbm, idx_v, *rest):
        bufs = rest[:NBUF]
        gsems = rest[NBUF:2 * NBUF]
        ssems = rest[2 * NBUF:3 * NBUF]

        cid = lax.axis_index("c")
        sid = lax.axis_index("s")
        wid = sid * NC + cid
        base = wid * ch_per_w

        pltpu.sync_copy(idx_hbm.at[pl.ds(base, ch_per_w)], idx_v)

        def gather(b, lc):
            return pltpu.make_async_copy(
                table_hbm.at[idx_v.at[lc]], bufs[b], gsems[b])

        def scatter(b, lc):
            row0 = pl.multiple_of((base + lc) * CHUNK, CHUNK)
            return pltpu.make_async_copy(
                bufs[b], out_hbm.at[pl.ds(row0, CHUNK)], ssems[b])

        for b in range(NBUF):
            gather(b, b).start()

        def body(t, carry):
            for b in range(NBUF):
                lc = t * NBUF + b
                gather(b, lc).wait()
                scatter(b, lc).start()
            for b in range(NBUF):
                lc = t * NBUF + b
                scatter(b, lc).wait()
                gather(b, jnp.minimum(lc + NBUF, ch_per_w - 1)).start()
            return carry

        lax.fori_loop(0, n_outer, body, 0)

        for b in range(NBUF):
            gather(b, ch_per_w - 1).wait()

    return emb


def kernel(table, expression_values):
    b, g = expression_values.shape
    d = table.shape[1]
    n = b * g
    idx2 = expression_values.reshape(n // CHUNK, CHUNK)
    out = _make_emb(n)(table, idx2)
    return out.reshape(b, g, d)


# speedup vs baseline: 10.4907x; 2.3598x over previous
import functools

import jax
import jax.numpy as jnp
from jax import lax
from jax.experimental import pallas as pl
from jax.experimental.pallas import tpu as pltpu
from jax.experimental.pallas import tpu_sc as plsc

D = 128
CHUNK = 128
NBUF = 2
NC = 2
NS = 16


@functools.cache
def _make_emb(n_rows, vocab):
    n_chunks = n_rows // CHUNK
    nw = NC * NS
    ch_per_w = n_chunks // nw
    n_outer = ch_per_w // NBUF
    assert n_chunks % nw == 0 and ch_per_w % NBUF == 0

    mesh = plsc.VectorSubcoreMesh(core_axis_name="c", subcore_axis_name="s")

    @functools.partial(
        pl.kernel,
        out_type=jax.ShapeDtypeStruct((n_rows, D), jnp.float32),
        mesh=mesh,
        scratch_types=[
            pltpu.VMEM_SHARED((vocab, D), jnp.float32),
            pltpu.VMEM((ch_per_w, CHUNK), jnp.int32),
            *[pltpu.VMEM((CHUNK, D), jnp.float32) for _ in range(NBUF)],
            *[pltpu.SemaphoreType.DMA for _ in range(2 * NBUF)],
        ],
    )
    def emb(table_hbm, idx_hbm, out_hbm, table_v, idx_v, *rest):
        bufs = rest[:NBUF]
        gsems = rest[NBUF:2 * NBUF]
        ssems = rest[2 * NBUF:3 * NBUF]

        cid = lax.axis_index("c")
        sid = lax.axis_index("s")
        wid = sid * NC + cid
        base = wid * ch_per_w

        @pl.when(sid == 0)
        def _():
            pltpu.sync_copy(table_hbm, table_v)

        pltpu.sync_copy(idx_hbm.at[pl.ds(base, ch_per_w)], idx_v)
        plsc.subcore_barrier()

        def gather(b, lc):
            return pltpu.make_async_copy(
                table_v.at[idx_v.at[lc]], bufs[b], gsems[b])

        def scatter(b, lc):
            row0 = pl.multiple_of((base + lc) * CHUNK, CHUNK)
            return pltpu.make_async_copy(
                bufs[b], out_hbm.at[pl.ds(row0, CHUNK)], ssems[b])

        for b in range(NBUF):
            gather(b, b).start()

        def body(t, carry):
            for b in range(NBUF):
                lc = t * NBUF + b
                gather(b, lc).wait()
                scatter(b, lc).start()
            for b in range(NBUF):
                lc = t * NBUF + b
                scatter(b, lc).wait()
                gather(b, jnp.minimum(lc + NBUF, ch_per_w - 1)).start()
            return carry

        lax.fori_loop(0, n_outer, body, 0)

        for b in range(NBUF):
            gather(b, ch_per_w - 1).wait()

    return emb


def kernel(table, expression_values):
    b, g = expression_values.shape
    d = table.shape[1]
    n = b * g
    idx2 = expression_values.reshape(n // CHUNK, CHUNK)
    out = _make_emb(n, table.shape[0])(table, idx2)
    return out.reshape(b, g, d)


# speedup vs baseline: 15.5584x; 1.4831x over previous
import functools

import jax
import jax.numpy as jnp
from jax import lax
from jax.experimental import pallas as pl
from jax.experimental.pallas import tpu as pltpu
from jax.experimental.pallas import tpu_sc as plsc

D = 128
CHUNK = 128
NBUF = 4
NC = 2
NS = 16


@functools.cache
def _make_emb(n_rows, vocab):
    n_chunks = n_rows // CHUNK
    nw = NC * NS
    ch_per_w = n_chunks // nw
    n_outer = ch_per_w // NBUF
    assert n_chunks % nw == 0 and ch_per_w % NBUF == 0

    mesh = plsc.VectorSubcoreMesh(core_axis_name="c", subcore_axis_name="s")

    @functools.partial(
        pl.kernel,
        out_type=jax.ShapeDtypeStruct((n_rows, D), jnp.float32),
        mesh=mesh,
        scratch_types=[
            pltpu.VMEM_SHARED((vocab, D), jnp.float32),
            pltpu.VMEM((ch_per_w, CHUNK), jnp.int32),
            *[pltpu.VMEM((CHUNK, D), jnp.float32) for _ in range(NBUF)],
            *[pltpu.SemaphoreType.DMA for _ in range(2 * NBUF)],
        ],
    )
    def emb(table_hbm, idx_hbm, out_hbm, table_v, idx_v, *rest):
        bufs = rest[:NBUF]
        gsems = rest[NBUF:2 * NBUF]
        ssems = rest[2 * NBUF:3 * NBUF]

        cid = lax.axis_index("c")
        sid = lax.axis_index("s")
        wid = sid * NC + cid
        base = wid * ch_per_w

        @pl.when(sid == 0)
        def _():
            pltpu.sync_copy(table_hbm, table_v)

        pltpu.sync_copy(idx_hbm.at[pl.ds(base, ch_per_w)], idx_v)
        plsc.subcore_barrier()

        def gather(b, lc):
            return pltpu.make_async_copy(
                table_v.at[idx_v.at[lc]], bufs[b], gsems[b])

        def scatter(b, lc):
            row0 = pl.multiple_of((base + lc) * CHUNK, CHUNK)
            return pltpu.make_async_copy(
                bufs[b], out_hbm.at[pl.ds(row0, CHUNK)], ssems[b])

        for b in range(NBUF):
            gather(b, b).start()

        def body(t, carry):
            for b in range(NBUF):
                lc = t * NBUF + b
                gather(b, lc).wait()
                scatter(b, lc).start()
            for b in range(NBUF):
                lc = t * NBUF + b
                scatter(b, lc).wait()
                gather(b, jnp.minimum(lc + NBUF, ch_per_w - 1)).start()
            return carry

        lax.fori_loop(0, n_outer, body, 0)

        for b in range(NBUF):
            gather(b, ch_per_w - 1).wait()

    return emb


def kernel(table, expression_values):
    b, g = expression_values.shape
    d = table.shape[1]
    n = b * g
    idx2 = expression_values.reshape(n // CHUNK, CHUNK)
    out = _make_emb(n, table.shape[0])(table, idx2)
    return out.reshape(b, g, d)
